# Initial kernel scaffold; baseline (speedup 1.0000x reference)
import jax
import jax.numpy as jnp
from jax.experimental import pallas as pl


def kernel(species, edge_src, edge_dst, distances, vec, polarizability, electric_field):
    raise NotImplementedError("write your pallas kernel here")


# speedup vs baseline: 55.3081x; 55.3081x over previous
import jax
import jax.numpy as jnp
from jax import lax
from jax.experimental import pallas as pl
from jax.experimental.pallas import tpu as pltpu
from jax.experimental.pallas import tpu_sc as plsc

BOHR = 0.52917721067
DAMP = 0.39

NC, NS, LANES = 2, 16, 16
NW = NC * NS
NPT = 6256
NP = NPT * NS
BATCH = 80
KB = 25
CHUNK = BATCH * KB
GROUPS = CHUNK // LANES

_f32 = jnp.float32
_i32 = jnp.int32


def _mesh():
    return plsc.VectorSubcoreMesh(core_axis_name="c", subcore_axis_name="s",
                                  num_cores=NC, num_subcores=NS)


def _coef_body(dist_hbm, srcr_hbm, dstr_hbm, isp_hbm, a_hbm, b_hbm,
               isp_s, src_t, dst_t, dist_t, gs_t, gd_t, a_t, b_t, sem):
    c = lax.axis_index("c")
    s = lax.axis_index("s")
    wid = c * NS + s
    n_edges = dist_hbm.shape[0]
    epw = n_edges // NW
    nch = epw // CHUNK
    pltpu.sync_copy(isp_hbm.at[pl.ds(s * NPT, NPT)], isp_s.at[pl.ds(s * NPT, NPT)])
    plsc.subcore_barrier()

    def chunk_body(k, carry):
        eb = wid * epw + k * CHUNK
        row0 = eb // BATCH
        pltpu.sync_copy(srcr_hbm.at[pl.ds(row0, KB), :], src_t)
        pltpu.sync_copy(dstr_hbm.at[pl.ds(row0, KB), :], dst_t)
        pltpu.sync_copy(dist_hbm.at[pl.ds(eb, CHUNK)], dist_t)
        descs = []
        for j in range(KB):
            descs.append(pltpu.async_copy(
                isp_s.at[src_t.at[j]], gs_t.at[pl.ds(j * BATCH, BATCH)], sem))
            descs.append(pltpu.async_copy(
                isp_s.at[dst_t.at[j]], gd_t.at[pl.ds(j * BATCH, BATCH)], sem))
        for d in descs:
            d.wait()

        def grp(i, carry2):
            w = pl.ds(i * LANES, LANES)
            r = dist_t[w] * (1.0 / BOHR)
            r2 = r * r
            r3 = r2 * r
            r5 = r3 * r2
            u3 = r3 * gs_t[w] * gd_t[w]
            au3 = DAMP * u3
            e = jnp.exp(-au3)
            lam3 = 1.0 - e
            lam5 = 1.0 - (1.0 + au3) * e
            b_t[w] = lam3 / r3
            a_t[w] = (3.0 / (BOHR * BOHR)) * lam5 / r5
            return carry2

        lax.fori_loop(0, GROUPS, grp, 0)
        pltpu.sync_copy(a_t, a_hbm.at[pl.ds(eb, CHUNK)])
        pltpu.sync_copy(b_t, b_hbm.at[pl.ds(eb, CHUNK)])
        return carry

    lax.fori_loop(0, nch, chunk_body, 0)


def _coef_call(distances, srcr, dstr, isp):
    n_edges = distances.shape[0]
    return pl.kernel(
        _coef_body,
        out_type=(jax.ShapeDtypeStruct((n_edges,), _f32),
                  jax.ShapeDtypeStruct((n_edges,), _f32)),
        mesh=_mesh(),
        compiler_params=pltpu.CompilerParams(use_tc_tiling_on_sc=False),
        scratch_types=[
            pltpu.VMEM_SHARED((NP,), _f32),
            pltpu.VMEM((KB, BATCH), _i32),
            pltpu.VMEM((KB, BATCH), _i32),
            pltpu.VMEM((CHUNK,), _f32),
            pltpu.VMEM((CHUNK,), _f32),
            pltpu.VMEM((CHUNK,), _f32),
            pltpu.VMEM((CHUNK,), _f32),
            pltpu.VMEM((CHUNK,), _f32),
            pltpu.SemaphoreType.DMA,
        ],
    )(distances, srcr, dstr, isp)


def _mv_body(px_hbm, py_hbm, pz_hbm, srcr_hbm, dstr_hbm,
             vx_hbm, vy_hbm, vz_hbm, a_hbm, b_hbm,
             hdx_hbm, hdy_hbm, hdz_hbm,
             partx_hbm, party_hbm, partz_hbm,
             px_s, py_s, pz_s, ax_s, ay_s, az_s, src_t, dst_t,
             vx_t, vy_t, vz_t, a_t, b_t,
             gx_t, gy_t, gz_t, ox_t, oy_t, oz_t, gsem, ssem):
    c = lax.axis_index("c")
    s = lax.axis_index("s")
    wid = c * NS + s
    n_edges = vx_hbm.shape[0]
    epw = n_edges // NW
    nch = epw // CHUNK
    nsl = pl.ds(s * NPT, NPT)
    pltpu.sync_copy(px_hbm.at[nsl], px_s.at[nsl])
    pltpu.sync_copy(py_hbm.at[nsl], py_s.at[nsl])
    pltpu.sync_copy(pz_hbm.at[nsl], pz_s.at[nsl])
    pltpu.sync_copy(hdx_hbm.at[nsl], ax_s.at[nsl])
    pltpu.sync_copy(hdy_hbm.at[nsl], ay_s.at[nsl])
    pltpu.sync_copy(hdz_hbm.at[nsl], az_s.at[nsl])
    plsc.subcore_barrier()

    def chunk_body(k, carry):
        eb = wid * epw + k * CHUNK
        row0 = eb // BATCH
        pltpu.sync_copy(srcr_hbm.at[pl.ds(row0, KB), :], src_t)
        pltpu.sync_copy(dstr_hbm.at[pl.ds(row0, KB), :], dst_t)
        pltpu.sync_copy(vx_hbm.at[pl.ds(eb, CHUNK)], vx_t)
        pltpu.sync_copy(vy_hbm.at[pl.ds(eb, CHUNK)], vy_t)
        pltpu.sync_copy(vz_hbm.at[pl.ds(eb, CHUNK)], vz_t)
        pltpu.sync_copy(a_hbm.at[pl.ds(eb, CHUNK)], a_t)
        pltpu.sync_copy(b_hbm.at[pl.ds(eb, CHUNK)], b_t)
        descs = []
        for j in range(KB):
            w = pl.ds(j * BATCH, BATCH)
            descs.append(pltpu.async_copy(px_s.at[dst_t.at[j]], gx_t.at[w], gsem))
            descs.append(pltpu.async_copy(py_s.at[dst_t.at[j]], gy_t.at[w], gsem))
            descs.append(pltpu.async_copy(pz_s.at[dst_t.at[j]], gz_t.at[w], gsem))
        for d in descs:
            d.wait()

        def grp(i, carry2):
            w = pl.ds(i * LANES, LANES)
            px = gx_t[w]
            py = gy_t[w]
            pz = gz_t[w]
            vx = vx_t[w]
            vy = vy_t[w]
            vz = vz_t[w]
            dot = vx * px + vy * py + vz * pz
            sca = a_t[w] * dot
            bb = b_t[w]
            ox_t[w] = sca * vx - bb * px
            oy_t[w] = sca * vy - bb * py
            oz_t[w] = sca * vz - bb * pz
            return carry2

        lax.fori_loop(0, GROUPS, grp, 0)
        sdescs = []
        for j in range(KB):
            w = pl.ds(j * BATCH, BATCH)
            sdescs.append(pltpu.async_copy(
                ox_t.at[w], ax_s.at[src_t.at[j]], ssem, add=True))
            sdescs.append(pltpu.async_copy(
                oy_t.at[w], ay_s.at[src_t.at[j]], ssem, add=True))
            sdescs.append(pltpu.async_copy(
                oz_t.at[w], az_s.at[src_t.at[j]], ssem, add=True))
        for d in sdescs:
            d.wait()
        return carry

    lax.fori_loop(0, nch, chunk_body, 0)
    plsc.subcore_barrier()
    pltpu.sync_copy(ax_s.at[nsl], partx_hbm.at[c, nsl])
    pltpu.sync_copy(ay_s.at[nsl], party_hbm.at[c, nsl])
    pltpu.sync_copy(az_s.at[nsl], partz_hbm.at[c, nsl])


def _mv_call(px, py, pz, srcr, dstr, vx, vy, vz, a, b, hdx, hdy, hdz):
    return pl.kernel(
        _mv_body,
        out_type=(jax.ShapeDtypeStruct((NC, NP), _f32),
                  jax.ShapeDtypeStruct((NC, NP), _f32),
                  jax.ShapeDtypeStruct((NC, NP), _f32)),
        mesh=_mesh(),
        compiler_params=pltpu.CompilerParams(use_tc_tiling_on_sc=False),
        scratch_types=[
            pltpu.VMEM_SHARED((NP,), _f32),
            pltpu.VMEM_SHARED((NP,), _f32),
            pltpu.VMEM_SHARED((NP,), _f32),
            pltpu.VMEM_SHARED((NP,), _f32),
            pltpu.VMEM_SHARED((NP,), _f32),
            pltpu.VMEM_SHARED((NP,), _f32),
            pltpu.VMEM((KB, BATCH), _i32),
            pltpu.VMEM((KB, BATCH), _i32),
            pltpu.VMEM((CHUNK,), _f32),
            pltpu.VMEM((CHUNK,), _f32),
            pltpu.VMEM((CHUNK,), _f32),
            pltpu.VMEM((CHUNK,), _f32),
            pltpu.VMEM((CHUNK,), _f32),
            pltpu.VMEM((CHUNK,), _f32),
            pltpu.VMEM((CHUNK,), _f32),
            pltpu.VMEM((CHUNK,), _f32),
            pltpu.VMEM((CHUNK,), _f32),
            pltpu.VMEM((CHUNK,), _f32),
            pltpu.VMEM((CHUNK,), _f32),
            pltpu.SemaphoreType.DMA,
            pltpu.SemaphoreType.DMA,
        ],
    )(px, py, pz, srcr, dstr, vx, vy, vz, a, b, hdx, hdy, hdz)


def kernel(species, edge_src, edge_dst, distances, vec, polarizability,
           electric_field):
    n_nodes = species.shape[0]
    n_edges = edge_src.shape[0]
    pol_b = polarizability.astype(_f32) / (BOHR ** 3)
    isp = jnp.concatenate(
        [lax.rsqrt(pol_b), jnp.ones((NP - n_nodes,), _f32)])
    tii = jnp.concatenate([1.0 / pol_b, jnp.zeros((NP - n_nodes,), _f32)])
    half_tii = 0.5 * tii
    srcr = edge_src.astype(_i32).reshape(n_edges // BATCH, BATCH)
    dstr = edge_dst.astype(_i32).reshape(n_edges // BATCH, BATCH)
    vx, vy, vz = [jnp.reshape(col, (n_edges,))
                  for col in jnp.split(vec, 3, axis=1)]
    a, b = _coef_call(distances.astype(_f32), srcr, dstr, isp)
    ef3 = electric_field.reshape(n_nodes, 3)
    pad = (0, NP - n_nodes)
    bx = jnp.pad(ef3[:, 0], pad)
    by = jnp.pad(ef3[:, 1], pad)
    bz = jnp.pad(ef3[:, 2], pad)

    def matvec(x):
        x1, x2, x3 = x
        parts = _mv_call(x1, x2, x3, srcr, dstr, vx, vy, vz, a, b,
                         half_tii * x1, half_tii * x2, half_tii * x3)
        return tuple(p[0] + p[1] for p in parts)

    mu = jax.scipy.sparse.linalg.cg(matvec, (bx, by, bz), maxiter=150)[0]
    mu = lax.stop_gradient(mu)
    tmu3 = matvec(mu)
    pol_energy = sum(((0.5 * t - bc) * m)
                     for t, bc, m in zip(tmu3, (bx, by, bz), mu))
    induced = jnp.stack(mu, axis=1)[:n_nodes] * BOHR
    tmu = jnp.stack(tmu3, axis=1)[:n_nodes]
    return (pol_energy[:n_nodes], induced, tmu, ef3)


# speedup vs baseline: 83.5026x; 1.5098x over previous
import jax
import jax.numpy as jnp
from jax import lax
from jax.experimental import pallas as pl
from jax.experimental.pallas import tpu as pltpu
from jax.experimental.pallas import tpu_sc as plsc

BOHR = 0.52917721067
DAMP = 0.39

NC, NS, LANES = 2, 16, 16
NW = NC * NS
NPT = 6256
NP = NPT * NS
BATCH = 80
KB = 25
CHUNK = BATCH * KB
GROUPS = CHUNK // LANES

_f32 = jnp.float32
_i32 = jnp.int32


def _mesh():
    return plsc.VectorSubcoreMesh(core_axis_name="c", subcore_axis_name="s",
                                  num_cores=NC, num_subcores=NS)


def _coef_body(dist_hbm, srcr_hbm, dstr_hbm, isp_hbm, a_hbm, b_hbm,
               isp_s, src_t, dst_t, dist_t, gs_t, gd_t, a_t, b_t, sem):
    c = lax.axis_index("c")
    s = lax.axis_index("s")
    wid = c * NS + s
    n_edges = dist_hbm.shape[0]
    epw = n_edges // NW
    nch = epw // CHUNK
    pltpu.sync_copy(isp_hbm.at[pl.ds(s * NPT, NPT)], isp_s.at[pl.ds(s * NPT, NPT)])
    plsc.subcore_barrier()

    def chunk_body(k, carry):
        eb = wid * epw + k * CHUNK
        row0 = eb // BATCH
        pltpu.sync_copy(srcr_hbm.at[pl.ds(row0, KB), :], src_t)
        pltpu.sync_copy(dstr_hbm.at[pl.ds(row0, KB), :], dst_t)
        pltpu.sync_copy(dist_hbm.at[pl.ds(eb, CHUNK)], dist_t)
        descs = []
        for j in range(KB):
            descs.append(pltpu.async_copy(
                isp_s.at[src_t.at[j]], gs_t.at[pl.ds(j * BATCH, BATCH)], sem))
            descs.append(pltpu.async_copy(
                isp_s.at[dst_t.at[j]], gd_t.at[pl.ds(j * BATCH, BATCH)], sem))
        for d in descs:
            d.wait()

        def grp(i, carry2):
            w = pl.ds(i * LANES, LANES)
            r = dist_t[w] * (1.0 / BOHR)
            r2 = r * r
            r3 = r2 * r
            r5 = r3 * r2
            u3 = r3 * gs_t[w] * gd_t[w]
            au3 = DAMP * u3
            e = jnp.exp(-au3)
            lam3 = 1.0 - e
            lam5 = 1.0 - (1.0 + au3) * e
            b_t[w] = lam3 / r3
            a_t[w] = (3.0 / (BOHR * BOHR)) * lam5 / r5
            return carry2

        lax.fori_loop(0, GROUPS, grp, 0)
        pltpu.sync_copy(a_t, a_hbm.at[pl.ds(eb, CHUNK)])
        pltpu.sync_copy(b_t, b_hbm.at[pl.ds(eb, CHUNK)])
        return carry

    lax.fori_loop(0, nch, chunk_body, 0)


def _coef_call(distances, srcr, dstr, isp):
    n_edges = distances.shape[0]
    return pl.kernel(
        _coef_body,
        out_type=(jax.ShapeDtypeStruct((n_edges,), _f32),
                  jax.ShapeDtypeStruct((n_edges,), _f32)),
        mesh=_mesh(),
        compiler_params=pltpu.CompilerParams(use_tc_tiling_on_sc=False),
        scratch_types=[
            pltpu.VMEM_SHARED((NP,), _f32),
            pltpu.VMEM((KB, BATCH), _i32),
            pltpu.VMEM((KB, BATCH), _i32),
            pltpu.VMEM((CHUNK,), _f32),
            pltpu.VMEM((CHUNK,), _f32),
            pltpu.VMEM((CHUNK,), _f32),
            pltpu.VMEM((CHUNK,), _f32),
            pltpu.VMEM((CHUNK,), _f32),
            pltpu.SemaphoreType.DMA,
        ],
    )(distances, srcr, dstr, isp)


def _mv_body(px_hbm, py_hbm, pz_hbm, srcr_hbm, dstr_hbm,
             vx_hbm, vy_hbm, vz_hbm, a_hbm, b_hbm,
             hdx_hbm, hdy_hbm, hdz_hbm,
             partx_hbm, party_hbm, partz_hbm,
             px_s, py_s, pz_s, ax_s, ay_s, az_s,
             srcA, srcB, dst_t,
             vx_t, vy_t, vz_t, a_t, b_t,
             gx_t, gy_t, gz_t,
             oxA, oyA, ozA, oxB, oyB, ozB, insem, gsem, ssem):
    c = lax.axis_index("c")
    s = lax.axis_index("s")
    wid = c * NS + s
    n_edges = vx_hbm.shape[0]
    epw = n_edges // NW
    nch = epw // CHUNK
    nsl = pl.ds(s * NPT, NPT)
    pltpu.sync_copy(px_hbm.at[nsl], px_s.at[nsl])
    pltpu.sync_copy(py_hbm.at[nsl], py_s.at[nsl])
    pltpu.sync_copy(pz_hbm.at[nsl], pz_s.at[nsl])
    pltpu.sync_copy(hdx_hbm.at[nsl], ax_s.at[nsl])
    pltpu.sync_copy(hdy_hbm.at[nsl], ay_s.at[nsl])
    pltpu.sync_copy(hdz_hbm.at[nsl], az_s.at[nsl])
    plsc.subcore_barrier()
    zf = jnp.zeros((LANES,), _f32)

    def emit_in(k, src_t):
        eb = wid * epw + k * CHUNK
        row0 = eb // BATCH
        d = [pltpu.async_copy(srcr_hbm.at[pl.ds(row0, KB), :], src_t, insem),
             pltpu.async_copy(dstr_hbm.at[pl.ds(row0, KB), :], dst_t, insem),
             pltpu.async_copy(vx_hbm.at[pl.ds(eb, CHUNK)], vx_t, insem),
             pltpu.async_copy(vy_hbm.at[pl.ds(eb, CHUNK)], vy_t, insem),
             pltpu.async_copy(vz_hbm.at[pl.ds(eb, CHUNK)], vz_t, insem),
             pltpu.async_copy(a_hbm.at[pl.ds(eb, CHUNK)], a_t, insem),
             pltpu.async_copy(b_hbm.at[pl.ds(eb, CHUNK)], b_t, insem)]
        for x in d:
            x.wait()

    def _bslice(buf, j):
        return buf.at[pl.ds(pl.multiple_of(j * BATCH, 8), BATCH)]

    def emit_gather():
        def gi(j, c2):
            pltpu.async_copy(px_s.at[dst_t.at[j]], _bslice(gx_t, j), gsem)
            pltpu.async_copy(py_s.at[dst_t.at[j]], _bslice(gy_t, j), gsem)
            pltpu.async_copy(pz_s.at[dst_t.at[j]], _bslice(gz_t, j), gsem)
            return c2

        lax.fori_loop(0, KB, gi, 0)

        def gw(j, c2):
            pltpu.make_async_copy(px_s.at[dst_t.at[j]], _bslice(gx_t, j), gsem).wait()
            pltpu.make_async_copy(py_s.at[dst_t.at[j]], _bslice(gy_t, j), gsem).wait()
            pltpu.make_async_copy(pz_s.at[dst_t.at[j]], _bslice(gz_t, j), gsem).wait()
            return c2

        lax.fori_loop(0, KB, gw, 0)

    def emit_compute(ox_t, oy_t, oz_t):
        def grp(i, carry2):
            w = pl.ds(i * LANES, LANES)
            px = gx_t[w]
            py = gy_t[w]
            pz = gz_t[w]
            vx = vx_t[w]
            vy = vy_t[w]
            vz = vz_t[w]
            dot = vx * px + vy * py + vz * pz
            sca = a_t[w] * dot
            bb = b_t[w]
            ox_t[w] = sca * vx - bb * px
            oy_t[w] = sca * vy - bb * py
            oz_t[w] = sca * vz - bb * pz
            return carry2

        lax.fori_loop(0, GROUPS, grp, 0)

    def emit_scatter_issue(src_t, ox_t, oy_t, oz_t):
        def si(j, c2):
            pltpu.async_copy(_bslice(ox_t, j), ax_s.at[src_t.at[j]], ssem,
                             add=True)
            pltpu.async_copy(_bslice(oy_t, j), ay_s.at[src_t.at[j]], ssem,
                             add=True)
            pltpu.async_copy(_bslice(oz_t, j), az_s.at[src_t.at[j]], ssem,
                             add=True)
            return c2

        lax.fori_loop(0, KB, si, 0)

    def emit_scatter_wait(src_t, ox_t, oy_t, oz_t):
        def sw(j, c2):
            pltpu.make_async_copy(_bslice(ox_t, j), ax_s.at[src_t.at[j]],
                                  ssem).wait()
            pltpu.make_async_copy(_bslice(oy_t, j), ay_s.at[src_t.at[j]],
                                  ssem).wait()
            pltpu.make_async_copy(_bslice(oz_t, j), az_s.at[src_t.at[j]],
                                  ssem).wait()
            return c2

        lax.fori_loop(0, KB, sw, 0)

    def do_chunk(k, cur, prev):
        src_c, ox_c, oy_c, oz_c = cur
        src_p, ox_p, oy_p, oz_p = prev
        emit_in(k, src_c)
        emit_gather()
        emit_compute(ox_c, oy_c, oz_c)
        emit_scatter_wait(src_p, ox_p, oy_p, oz_p)
        emit_scatter_issue(src_c, ox_c, oy_c, oz_c)

    bufsA = (srcA, oxA, oyA, ozA)
    bufsB = (srcB, oxB, oyB, ozB)

    pltpu.sync_copy(srcr_hbm.at[pl.ds(wid * (epw // BATCH), KB), :], srcB)

    def zero_grp(i, c2):
        w = pl.ds(i * LANES, LANES)
        oxB[w] = zf
        oyB[w] = zf
        ozB[w] = zf
        return c2

    lax.fori_loop(0, GROUPS, zero_grp, 0)
    emit_scatter_issue(srcB, oxB, oyB, ozB)

    def pair_body(t, carry):
        do_chunk(2 * t, bufsA, bufsB)
        do_chunk(2 * t + 1, bufsB, bufsA)
        return carry

    lax.fori_loop(0, (nch - 1) // 2, pair_body, 0)
    do_chunk(nch - 1, bufsA, bufsB)
    emit_scatter_wait(srcA, oxA, oyA, ozA)
    plsc.subcore_barrier()
    pltpu.sync_copy(ax_s.at[nsl], partx_hbm.at[c, nsl])
    pltpu.sync_copy(ay_s.at[nsl], party_hbm.at[c, nsl])
    pltpu.sync_copy(az_s.at[nsl], partz_hbm.at[c, nsl])


def _mv_call(px, py, pz, srcr, dstr, vx, vy, vz, a, b, hdx, hdy, hdz):
    return pl.kernel(
        _mv_body,
        out_type=(jax.ShapeDtypeStruct((NC, NP), _f32),
                  jax.ShapeDtypeStruct((NC, NP), _f32),
                  jax.ShapeDtypeStruct((NC, NP), _f32)),
        mesh=_mesh(),
        compiler_params=pltpu.CompilerParams(use_tc_tiling_on_sc=False),
        scratch_types=[
            pltpu.VMEM_SHARED((NP,), _f32),
            pltpu.VMEM_SHARED((NP,), _f32),
            pltpu.VMEM_SHARED((NP,), _f32),
            pltpu.VMEM_SHARED((NP,), _f32),
            pltpu.VMEM_SHARED((NP,), _f32),
            pltpu.VMEM_SHARED((NP,), _f32),
            pltpu.VMEM((KB, BATCH), _i32),
            pltpu.VMEM((KB, BATCH), _i32),
            pltpu.VMEM((KB, BATCH), _i32),
            pltpu.VMEM((CHUNK,), _f32),
            pltpu.VMEM((CHUNK,), _f32),
            pltpu.VMEM((CHUNK,), _f32),
            pltpu.VMEM((CHUNK,), _f32),
            pltpu.VMEM((CHUNK,), _f32),
            pltpu.VMEM((CHUNK,), _f32),
            pltpu.VMEM((CHUNK,), _f32),
            pltpu.VMEM((CHUNK,), _f32),
            pltpu.VMEM((CHUNK,), _f32),
            pltpu.VMEM((CHUNK,), _f32),
            pltpu.VMEM((CHUNK,), _f32),
            pltpu.VMEM((CHUNK,), _f32),
            pltpu.VMEM((CHUNK,), _f32),
            pltpu.VMEM((CHUNK,), _f32),
            pltpu.SemaphoreType.DMA,
            pltpu.SemaphoreType.DMA,
            pltpu.SemaphoreType.DMA,
        ],
    )(px, py, pz, srcr, dstr, vx, vy, vz, a, b, hdx, hdy, hdz)


def kernel(species, edge_src, edge_dst, distances, vec, polarizability,
           electric_field):
    n_nodes = species.shape[0]
    n_edges = edge_src.shape[0]
    pol_b = polarizability.astype(_f32) / (BOHR ** 3)
    isp = jnp.concatenate(
        [lax.rsqrt(pol_b), jnp.ones((NP - n_nodes,), _f32)])
    tii = jnp.concatenate([1.0 / pol_b, jnp.zeros((NP - n_nodes,), _f32)])
    half_tii = 0.5 * tii
    srcr = edge_src.astype(_i32).reshape(n_edges // BATCH, BATCH)
    dstr = edge_dst.astype(_i32).reshape(n_edges // BATCH, BATCH)
    vx, vy, vz = [jnp.reshape(col, (n_edges,))
                  for col in jnp.split(vec, 3, axis=1)]
    a, b = _coef_call(distances.astype(_f32), srcr, dstr, isp)
    ef3 = electric_field.reshape(n_nodes, 3)
    pad = (0, NP - n_nodes)
    bx = jnp.pad(ef3[:, 0], pad)
    by = jnp.pad(ef3[:, 1], pad)
    bz = jnp.pad(ef3[:, 2], pad)

    def matvec(x):
        x1, x2, x3 = x
        parts = _mv_call(x1, x2, x3, srcr, dstr, vx, vy, vz, a, b,
                         half_tii * x1, half_tii * x2, half_tii * x3)
        return tuple(p[0] + p[1] for p in parts)

    mu = jax.scipy.sparse.linalg.cg(matvec, (bx, by, bz), maxiter=150)[0]
    mu = lax.stop_gradient(mu)
    tmu3 = matvec(mu)
    pol_energy = sum(((0.5 * t - bc) * m)
                     for t, bc, m in zip(tmu3, (bx, by, bz), mu))
    induced = jnp.stack(mu, axis=1)[:n_nodes] * BOHR
    tmu = jnp.stack(tmu3, axis=1)[:n_nodes]
    return (pol_energy[:n_nodes], induced, tmu, ef3)


# speedup vs baseline: 91.2286x; 1.0925x over previous
import jax
import jax.numpy as jnp
from jax import lax
from jax.experimental import pallas as pl
from jax.experimental.pallas import tpu as pltpu
from jax.experimental.pallas import tpu_sc as plsc

BOHR = 0.52917721067
DAMP = 0.39

NC, NS, LANES = 2, 16, 16
NW = NC * NS
NPT = 6256
NP = NPT * NS
BATCH = 80
KB = 25
CHUNK = BATCH * KB
GROUPS = CHUNK // LANES

_f32 = jnp.float32
_i32 = jnp.int32


def _mesh():
    return plsc.VectorSubcoreMesh(core_axis_name="c", subcore_axis_name="s",
                                  num_cores=NC, num_subcores=NS)


def _coef_body(dist_hbm, srcr_hbm, dstr_hbm, isp_hbm, a_hbm, b_hbm,
               isp_s, src_t, dst_t, dist_t, gs_t, gd_t, a_t, b_t, sem):
    c = lax.axis_index("c")
    s = lax.axis_index("s")
    wid = c * NS + s
    n_edges = dist_hbm.shape[0]
    epw = n_edges // NW
    nch = epw // CHUNK
    pltpu.sync_copy(isp_hbm.at[pl.ds(s * NPT, NPT)], isp_s.at[pl.ds(s * NPT, NPT)])
    plsc.subcore_barrier()

    def chunk_body(k, carry):
        eb = wid * epw + k * CHUNK
        row0 = eb // BATCH
        pltpu.sync_copy(srcr_hbm.at[pl.ds(row0, KB), :], src_t)
        pltpu.sync_copy(dstr_hbm.at[pl.ds(row0, KB), :], dst_t)
        pltpu.sync_copy(dist_hbm.at[pl.ds(eb, CHUNK)], dist_t)
        descs = []
        for j in range(KB):
            descs.append(pltpu.async_copy(
                isp_s.at[src_t.at[j]], gs_t.at[pl.ds(j * BATCH, BATCH)], sem))
            descs.append(pltpu.async_copy(
                isp_s.at[dst_t.at[j]], gd_t.at[pl.ds(j * BATCH, BATCH)], sem))
        for d in descs:
            d.wait()

        def grp(i, carry2):
            w = pl.ds(i * LANES, LANES)
            r = dist_t[w] * (1.0 / BOHR)
            r2 = r * r
            r3 = r2 * r
            r5 = r3 * r2
            u3 = r3 * gs_t[w] * gd_t[w]
            au3 = DAMP * u3
            e = jnp.exp(-au3)
            lam3 = 1.0 - e
            lam5 = 1.0 - (1.0 + au3) * e
            b_t[w] = lam3 / r3
            a_t[w] = (3.0 / (BOHR * BOHR)) * lam5 / r5
            return carry2

        lax.fori_loop(0, GROUPS, grp, 0)
        pltpu.sync_copy(a_t, a_hbm.at[pl.ds(eb, CHUNK)])
        pltpu.sync_copy(b_t, b_hbm.at[pl.ds(eb, CHUNK)])
        return carry

    lax.fori_loop(0, nch, chunk_body, 0)


def _coef_call(distances, srcr, dstr, isp):
    n_edges = distances.shape[0]
    return pl.kernel(
        _coef_body,
        out_type=(jax.ShapeDtypeStruct((n_edges,), _f32),
                  jax.ShapeDtypeStruct((n_edges,), _f32)),
        mesh=_mesh(),
        compiler_params=pltpu.CompilerParams(use_tc_tiling_on_sc=False),
        scratch_types=[
            pltpu.VMEM_SHARED((NP,), _f32),
            pltpu.VMEM((KB, BATCH), _i32),
            pltpu.VMEM((KB, BATCH), _i32),
            pltpu.VMEM((CHUNK,), _f32),
            pltpu.VMEM((CHUNK,), _f32),
            pltpu.VMEM((CHUNK,), _f32),
            pltpu.VMEM((CHUNK,), _f32),
            pltpu.VMEM((CHUNK,), _f32),
            pltpu.SemaphoreType.DMA,
        ],
    )(distances, srcr, dstr, isp)


def _mv_body(px_hbm, py_hbm, pz_hbm, srcr_hbm, dstr_hbm,
             vx_hbm, vy_hbm, vz_hbm, a_hbm, b_hbm,
             hdx_hbm, hdy_hbm, hdz_hbm,
             partx_hbm, party_hbm, partz_hbm,
             px_s, py_s, pz_s, ax_s, ay_s, az_s,
             src0, src1, src2, dst0, dst1,
             vxA, vyA, vzA, aA, bA, vxB, vyB, vzB, aB, bB,
             gxA, gyA, gzA, gxB, gyB, gzB,
             oxA, oyA, ozA, oxB, oyB, ozB, insem, gsem, ssem):
    c = lax.axis_index("c")
    s = lax.axis_index("s")
    wid = c * NS + s
    n_edges = vx_hbm.shape[0]
    epw = n_edges // NW
    nch = epw // CHUNK
    nsl = pl.ds(s * NPT, NPT)
    pltpu.sync_copy(px_hbm.at[nsl], px_s.at[nsl])
    pltpu.sync_copy(py_hbm.at[nsl], py_s.at[nsl])
    pltpu.sync_copy(pz_hbm.at[nsl], pz_s.at[nsl])
    pltpu.sync_copy(hdx_hbm.at[nsl], ax_s.at[nsl])
    pltpu.sync_copy(hdy_hbm.at[nsl], ay_s.at[nsl])
    pltpu.sync_copy(hdz_hbm.at[nsl], az_s.at[nsl])
    plsc.subcore_barrier()
    zf = jnp.zeros((LANES,), _f32)

    srcs = (src0, src1, src2)
    dsts = (dst0, dst1)
    coefs = ((vxA, vyA, vzA, aA, bA), (vxB, vyB, vzB, aB, bB))
    gs = ((gxA, gyA, gzA), (gxB, gyB, gzB))
    os_ = ((oxA, oyA, ozA), (oxB, oyB, ozB))

    def _ebrow(k):
        kk = jnp.minimum(k, nch - 1)
        eb = wid * epw + kk * CHUNK
        return eb, eb // BATCH

    def _in_descs(k, src_t, dst_t, coef):
        eb, row0 = _ebrow(k)
        vx_t, vy_t, vz_t, a_t, b_t = coef
        return [
            pltpu.make_async_copy(srcr_hbm.at[pl.ds(row0, KB), :], src_t, insem),
            pltpu.make_async_copy(dstr_hbm.at[pl.ds(row0, KB), :], dst_t, insem),
            pltpu.make_async_copy(vx_hbm.at[pl.ds(eb, CHUNK)], vx_t, insem),
            pltpu.make_async_copy(vy_hbm.at[pl.ds(eb, CHUNK)], vy_t, insem),
            pltpu.make_async_copy(vz_hbm.at[pl.ds(eb, CHUNK)], vz_t, insem),
            pltpu.make_async_copy(a_hbm.at[pl.ds(eb, CHUNK)], a_t, insem),
            pltpu.make_async_copy(b_hbm.at[pl.ds(eb, CHUNK)], b_t, insem),
        ]

    def in_issue(k, i):
        for d in _in_descs(k, srcs[i % 3], dsts[i % 2], coefs[i % 2]):
            d.start()

    def in_wait(k, i):
        for d in _in_descs(k, srcs[i % 3], dsts[i % 2], coefs[i % 2]):
            d.wait()

    def _bslice(buf, j):
        return buf.at[pl.ds(pl.multiple_of(j * BATCH, 8), BATCH)]

    def gather_issue(i):
        dst_t = dsts[i % 2]
        gx_t, gy_t, gz_t = gs[i % 2]

        def gi(j, c2):
            pltpu.async_copy(px_s.at[dst_t.at[j]], _bslice(gx_t, j), gsem)
            pltpu.async_copy(py_s.at[dst_t.at[j]], _bslice(gy_t, j), gsem)
            pltpu.async_copy(pz_s.at[dst_t.at[j]], _bslice(gz_t, j), gsem)
            return c2

        lax.fori_loop(0, KB, gi, 0)

    def gather_drain(i):
        dst_t = dsts[i % 2]
        gx_t, gy_t, gz_t = gs[i % 2]

        def gw(j, c2):
            pltpu.make_async_copy(px_s.at[dst_t.at[j]], _bslice(gx_t, j), gsem).wait()
            pltpu.make_async_copy(py_s.at[dst_t.at[j]], _bslice(gy_t, j), gsem).wait()
            pltpu.make_async_copy(pz_s.at[dst_t.at[j]], _bslice(gz_t, j), gsem).wait()
            return c2

        lax.fori_loop(0, KB, gw, 0)

    def compute(i):
        gx_t, gy_t, gz_t = gs[i % 2]
        vx_t, vy_t, vz_t, a_t, b_t = coefs[i % 2]
        ox_t, oy_t, oz_t = os_[i % 2]

        def grp(g, carry2):
            w = pl.ds(g * LANES, LANES)
            px = gx_t[w]
            py = gy_t[w]
            pz = gz_t[w]
            vx = vx_t[w]
            vy = vy_t[w]
            vz = vz_t[w]
            dot = vx * px + vy * py + vz * pz
            sca = a_t[w] * dot
            bb = b_t[w]
            ox_t[w] = sca * vx - bb * px
            oy_t[w] = sca * vy - bb * py
            oz_t[w] = sca * vz - bb * pz
            return carry2

        lax.fori_loop(0, GROUPS, grp, 0)

    def scatter_issue(i):
        src_t = srcs[i % 3]
        ox_t, oy_t, oz_t = os_[i % 2]

        def si(j, c2):
            pltpu.async_copy(_bslice(ox_t, j), ax_s.at[src_t.at[j]], ssem,
                             add=True)
            pltpu.async_copy(_bslice(oy_t, j), ay_s.at[src_t.at[j]], ssem,
                             add=True)
            pltpu.async_copy(_bslice(oz_t, j), az_s.at[src_t.at[j]], ssem,
                             add=True)
            return c2

        lax.fori_loop(0, KB, si, 0)

    def scatter_wait(i):
        src_t = srcs[i % 3]
        ox_t, oy_t, oz_t = os_[i % 2]

        def sw(j, c2):
            pltpu.make_async_copy(_bslice(ox_t, j), ax_s.at[src_t.at[j]],
                                  ssem).wait()
            pltpu.make_async_copy(_bslice(oy_t, j), ay_s.at[src_t.at[j]],
                                  ssem).wait()
            pltpu.make_async_copy(_bslice(oz_t, j), az_s.at[src_t.at[j]],
                                  ssem).wait()
            return c2

        lax.fori_loop(0, KB, sw, 0)

    def do_chunk(k, i, tail=False):
        gather_drain(i)
        in_wait(k + 1, i + 1)
        if not tail:
            gather_issue(i + 1)
        compute(i)
        scatter_wait(i - 1)
        scatter_issue(i)
        if not tail:
            in_issue(k + 2, i + 2)

    in_issue(0, 0)
    in_issue(1, 1)
    pltpu.sync_copy(srcr_hbm.at[pl.ds(wid * (epw // BATCH), KB), :], src2)

    def zero_grp(g, c2):
        w = pl.ds(g * LANES, LANES)
        oxB[w] = zf
        oyB[w] = zf
        ozB[w] = zf
        return c2

    lax.fori_loop(0, GROUPS, zero_grp, 0)
    scatter_issue(-1)
    in_wait(0, 0)
    gather_issue(0)

    def six_body(t, carry):
        for i in range(6):
            do_chunk(6 * t + i, i)
        return carry

    lax.fori_loop(0, (nch - 1) // 6, six_body, 0)
    do_chunk(nch - 1, 0, tail=True)
    scatter_wait(nch - 1)
    plsc.subcore_barrier()
    pltpu.sync_copy(ax_s.at[nsl], partx_hbm.at[c, nsl])
    pltpu.sync_copy(ay_s.at[nsl], party_hbm.at[c, nsl])
    pltpu.sync_copy(az_s.at[nsl], partz_hbm.at[c, nsl])


def _mv_call(px, py, pz, srcr, dstr, vx, vy, vz, a, b, hdx, hdy, hdz):
    return pl.kernel(
        _mv_body,
        out_type=(jax.ShapeDtypeStruct((NC, NP), _f32),
                  jax.ShapeDtypeStruct((NC, NP), _f32),
                  jax.ShapeDtypeStruct((NC, NP), _f32)),
        mesh=_mesh(),
        compiler_params=pltpu.CompilerParams(use_tc_tiling_on_sc=False),
        scratch_types=[
            pltpu.VMEM_SHARED((NP,), _f32),
            pltpu.VMEM_SHARED((NP,), _f32),
            pltpu.VMEM_SHARED((NP,), _f32),
            pltpu.VMEM_SHARED((NP,), _f32),
            pltpu.VMEM_SHARED((NP,), _f32),
            pltpu.VMEM_SHARED((NP,), _f32),
            pltpu.VMEM((KB, BATCH), _i32),
            pltpu.VMEM((KB, BATCH), _i32),
            pltpu.VMEM((KB, BATCH), _i32),
            pltpu.VMEM((KB, BATCH), _i32),
            pltpu.VMEM((KB, BATCH), _i32),
        ] + [pltpu.VMEM((CHUNK,), _f32)] * 22 + [
            pltpu.SemaphoreType.DMA,
            pltpu.SemaphoreType.DMA,
            pltpu.SemaphoreType.DMA,
        ],
    )(px, py, pz, srcr, dstr, vx, vy, vz, a, b, hdx, hdy, hdz)


def kernel(species, edge_src, edge_dst, distances, vec, polarizability,
           electric_field):
    n_nodes = species.shape[0]
    n_edges = edge_src.shape[0]
    pol_b = polarizability.astype(_f32) / (BOHR ** 3)
    isp = jnp.concatenate(
        [lax.rsqrt(pol_b), jnp.ones((NP - n_nodes,), _f32)])
    tii = jnp.concatenate([1.0 / pol_b, jnp.zeros((NP - n_nodes,), _f32)])
    half_tii = 0.5 * tii
    srcr = edge_src.astype(_i32).reshape(n_edges // BATCH, BATCH)
    dstr = edge_dst.astype(_i32).reshape(n_edges // BATCH, BATCH)
    vx, vy, vz = [jnp.reshape(col, (n_edges,))
                  for col in jnp.split(vec, 3, axis=1)]
    a, b = _coef_call(distances.astype(_f32), srcr, dstr, isp)
    ef3 = electric_field.reshape(n_nodes, 3)
    pad = (0, NP - n_nodes)
    bx = jnp.pad(ef3[:, 0], pad)
    by = jnp.pad(ef3[:, 1], pad)
    bz = jnp.pad(ef3[:, 2], pad)

    def matvec(x):
        x1, x2, x3 = x
        parts = _mv_call(x1, x2, x3, srcr, dstr, vx, vy, vz, a, b,
                         half_tii * x1, half_tii * x2, half_tii * x3)
        return tuple(p[0] + p[1] for p in parts)

    mu = jax.scipy.sparse.linalg.cg(matvec, (bx, by, bz), maxiter=150)[0]
    mu = lax.stop_gradient(mu)
    tmu3 = matvec(mu)
    pol_energy = sum(((0.5 * t - bc) * m)
                     for t, bc, m in zip(tmu3, (bx, by, bz), mu))
    induced = jnp.stack(mu, axis=1)[:n_nodes] * BOHR
    tmu = jnp.stack(tmu3, axis=1)[:n_nodes]
    return (pol_energy[:n_nodes], induced, tmu, ef3)
